# Initial kernel scaffold; baseline (speedup 1.0000x reference)
import jax
import jax.numpy as jnp
from jax.experimental import pallas as pl


def kernel(inputs, table):
    raise NotImplementedError("write your pallas kernel here")


# speedup vs baseline: 1.0987x; 1.0987x over previous
import functools

import jax
import jax.numpy as jnp
import numpy as np
from jax import lax
from jax.experimental import pallas as pl
from jax.experimental.pallas import tpu as pltpu
from jax.experimental.pallas import tpu_sc as plsc

_NUM_BINS = 1000000
_EMBED = 32

_NC, _NS, _L = 2, 16, 16
_NW = _NC * _NS

_B = 16384 * 50
_PER_W = _B // _NW
_CH = 1280
_NCH = _PER_W // _CH
_G = _CH // 128

_C1 = np.int32(np.uint32(2654435761).astype(np.int64) - (1 << 32))
_C2 = np.int32(np.uint32(2246822519).astype(np.int64) - (1 << 32))
_RCP = np.float32(16.0 / _NUM_BINS)


def _hash_mod(x):
    x = x * _C1
    x = x ^ lax.shift_right_logical(x, 16)
    x = x * _C2
    x = x ^ lax.shift_right_logical(x, 13)
    xs = lax.shift_right_logical(x, 4)
    q = (xs.astype(jnp.float32) * _RCP).astype(jnp.int32)
    r = x - q * np.int32(_NUM_BINS)
    r = jnp.where(r < 0, r + _NUM_BINS, r)
    r = jnp.where(r >= _NUM_BINS, r - _NUM_BINS, r)
    return r


def _body(in_hbm, table_hbm, out_hbm, in_v, idx_v, rows_v, sem):
    wid = lax.axis_index("s") * _NC + lax.axis_index("c")
    w_base = wid * _PER_W

    def chunk(c, carry):
        base = w_base + c * _CH
        pltpu.sync_copy(in_hbm.at[pl.ds(base, _CH)], in_v)

        descs = []
        for g in range(_G):
            for j in range(128 // _L):
                off = g * 128 + j * _L
                idx_v[g, pl.ds(j * _L, _L)] = _hash_mod(in_v[pl.ds(off, _L)])
            descs.append(
                pltpu.async_copy(
                    table_hbm.at[idx_v.at[g]],
                    rows_v.at[pl.ds(g * 128, 128)],
                    sem,
                )
            )
        for d in descs:
            d.wait()

        pltpu.sync_copy(rows_v, out_hbm.at[pl.ds(base, _CH)])
        return carry

    lax.fori_loop(0, _NCH, chunk, 0)


@jax.jit
def kernel(inputs, table):
    flat = inputs.reshape(_B)
    mesh = plsc.VectorSubcoreMesh(core_axis_name="c", subcore_axis_name="s")
    run = functools.partial(
        pl.kernel,
        mesh=mesh,
        out_type=jax.ShapeDtypeStruct((_B, _EMBED), jnp.float32),
        scratch_types=[
            pltpu.VMEM((_CH,), jnp.int32),
            pltpu.VMEM((_G, 128), jnp.int32),
            pltpu.VMEM((_CH, _EMBED), jnp.float32),
            pltpu.SemaphoreType.DMA,
        ],
        compiler_params=pltpu.CompilerParams(use_tc_tiling_on_sc=False),
    )(_body)
    out = run(flat, table)
    return out.reshape(inputs.shape[0], inputs.shape[1], _EMBED)
